# Initial kernel scaffold; baseline (speedup 1.0000x reference)
import jax
import jax.numpy as jnp
from jax.experimental import pallas as pl


def kernel(x0, i0, x1, C):
    raise NotImplementedError("write your pallas kernel here")


# speedup vs baseline: 14.4379x; 14.4379x over previous
import functools

import jax
import jax.numpy as jnp
from jax import lax
from jax.experimental import pallas as pl
from jax.experimental.pallas import tpu as pltpu
from jax.experimental.pallas import tpu_sc as plsc

N = 100000
Z = 100000
S = 4
U = 32
D = 3
F = S * U
B = 160
NBLK = N // B
NW = 32
BLKS_PER_W = (NBLK + NW - 1) // NW
L = 16


def _body(x0_hbm, i0_hbm, x1_hbm, cb_hbm, out_hbm,
          idx_v, g_v, x1_v, out_v, cb_v, sem_g, sem_x):
    wid = lax.axis_index("s") * 2 + lax.axis_index("c")

    pltpu.sync_copy(cb_hbm, cb_v)

    def do_block(t, _):
        blk = wid + t * NW

        @pl.when(blk < NBLK)
        def _():
            base = blk * B
            pltpu.sync_copy(i0_hbm.at[pl.ds(base, B)], idx_v)
            cp_g = pltpu.async_copy(x0_hbm.at[idx_v], g_v, sem_g)
            cp_x = pltpu.async_copy(x1_hbm.at[pl.ds(base, B)], x1_v, sem_x)
            cp_g.wait()
            cp_x.wait()

            cb = [[[cb_v[d, o, s, :] for s in range(S)] for o in range(S)]
                  for d in range(D)]

            def row(r, _):
                g = [g_v[r, pl.ds(j * L, L)] for j in range(F // L)]
                for o in range(S):
                    for h in range(U // L):
                        j = o * (U // L) + h
                        xo = x1_v[r, pl.ds(j * L, L)]
                        p = xo
                        acc = None
                        for d in range(D):
                            m = cb[d][o][0] * g[0 * (U // L) + h]
                            for s in range(1, S):
                                m = m + cb[d][o][s] * g[s * (U // L) + h]
                            term = m * p
                            acc = term if acc is None else acc + term
                            if d + 1 < D:
                                p = p * xo
                        out_v[r, pl.ds(j * L, L)] = acc
                return _

            lax.fori_loop(0, B, row, None)
            pltpu.sync_copy(out_v, out_hbm.at[pl.ds(base, B)])

        return _

    lax.fori_loop(0, BLKS_PER_W, do_block, None)


@jax.jit
def _run(x0, i0, x1, cb):
    mesh = plsc.VectorSubcoreMesh(core_axis_name="c", subcore_axis_name="s")
    fn = functools.partial(
        pl.kernel,
        mesh=mesh,
        out_type=jax.ShapeDtypeStruct((N, F), jnp.float32),
        scratch_types=[
            pltpu.VMEM((B,), jnp.int32),
            pltpu.VMEM((B, F), jnp.float32),
            pltpu.VMEM((B, F), jnp.float32),
            pltpu.VMEM((B, F), jnp.float32),
            pltpu.VMEM((D, S, S, L), jnp.float32),
            pltpu.SemaphoreType.DMA,
            pltpu.SemaphoreType.DMA,
        ],
    )(_body)
    return fn(x0, i0, x1, cb)


def kernel(x0, i0, x1, C):
    i0 = i0.astype(jnp.int32)
    cb = jnp.broadcast_to(C[:, :, :, None], (D, S, S, L)).astype(jnp.float32)
    return _run(x0, i0, x1, cb)
